# Initial kernel scaffold; baseline (speedup 1.0000x reference)
import functools

import jax
import jax.numpy as jnp
from jax import lax
from jax.experimental import pallas as pl
from jax.experimental.pallas import tpu as pltpu
from jax.experimental.pallas import tpu_sc as plsc

N = 10000
F = 128
E1 = 320000
E2 = 640000
NC = 2
NS = 16
NW = NC * NS
CHUNK = 80


def _spmm_body(z1, z2, s1, d1, w1, s2, d2, w2, zeros,
               out0, out1, acc, idx_v, dst_v, w_v, rows_v, sem):
  c = lax.axis_index("c")
  s = lax.axis_index("s")
  wid = c * NS + s

  @pl.when(s == 0)
  def _():
    pltpu.sync_copy(zeros, acc)

  plsc.subcore_barrier()

  def process(src_hbm, dst_hbm, ew_hbm, ztab, base, nchunks):
    def body(j, carry):
      off = pl.multiple_of(base + j * CHUNK, 8)
      pltpu.sync_copy(src_hbm.at[pl.ds(off, CHUNK)], idx_v)
      pltpu.sync_copy(dst_hbm.at[pl.ds(off, CHUNK)], dst_v)
      pltpu.sync_copy(ew_hbm.at[pl.ds(off, CHUNK)], w_v)
      pltpu.async_copy(ztab.at[idx_v], rows_v, sem).wait()

      def scale(i, carry2):
        wb = plsc.load_gather(w_v, [jnp.full((16,), i, jnp.int32)])
        for jj in range(8):
          rows_v[i, pl.ds(jj * 16, 16)] = rows_v[i, pl.ds(jj * 16, 16)] * wb
        return carry2

      lax.fori_loop(0, CHUNK, scale, 0)
      pltpu.sync_copy(rows_v, acc.at[dst_v], add=True)
      return carry

    lax.fori_loop(0, nchunks, body, 0)

  process(s1, d1, w1, z1, wid * (E1 // NW), (E1 // NW) // CHUNK)
  process(s2, d2, w2, z2, wid * (E2 // NW), (E2 // NW) // CHUNK)

  plsc.subcore_barrier()

  @pl.when((s == 0) & (c == 0))
  def _():
    pltpu.sync_copy(acc, out0)

  @pl.when((s == 0) & (c == 1))
  def _():
    pltpu.sync_copy(acc, out1)


_spmm = functools.partial(
    pl.kernel,
    out_type=[jax.ShapeDtypeStruct((N, F), jnp.float32),
              jax.ShapeDtypeStruct((N, F), jnp.float32)],
    mesh=plsc.VectorSubcoreMesh(core_axis_name="c", subcore_axis_name="s"),
    scratch_types=[
        pltpu.VMEM_SHARED((N, F), jnp.float32),
        pltpu.VMEM((CHUNK,), jnp.int32),
        pltpu.VMEM((CHUNK,), jnp.int32),
        pltpu.VMEM((CHUNK,), jnp.float32),
        pltpu.VMEM((CHUNK, F), jnp.float32),
        pltpu.SemaphoreType.DMA,
    ],
)(_spmm_body)


_RB = 1000


def _mm1_body(x_ref, w_ref, b_ref, o_ref):
  o_ref[...] = lax.dot_general(
      x_ref[...], w_ref[...], (((1,), (0,)), ((), ())),
      preferred_element_type=jnp.float32,
      precision=lax.Precision.HIGHEST) + b_ref[...]


def _mm3_body(a_ref, p0_ref, p1_ref, w_ref, b_ref, o_ref):
  h = a_ref[...] + p0_ref[...] + p1_ref[...]
  o_ref[...] = lax.dot_general(
      h, w_ref[...], (((1,), (0,)), ((), ())),
      preferred_element_type=jnp.float32,
      precision=lax.Precision.HIGHEST) + b_ref[...]


def _add3_body(a_ref, p0_ref, p1_ref, o_ref):
  o_ref[...] = a_ref[...] + p0_ref[...] + p1_ref[...]


def _mm1(x, w, b):
  return pl.pallas_call(
      _mm1_body,
      grid=(N // _RB,),
      in_specs=[
          pl.BlockSpec((_RB, F), lambda i: (i, 0)),
          pl.BlockSpec((F, 3 * F), lambda i: (0, 0)),
          pl.BlockSpec((1, 3 * F), lambda i: (0, 0)),
      ],
      out_specs=pl.BlockSpec((_RB, 3 * F), lambda i: (i, 0)),
      out_shape=jax.ShapeDtypeStruct((N, 3 * F), jnp.float32),
  )(x, w, b)


def _mm3(a, p0, p1, w, b):
  return pl.pallas_call(
      _mm3_body,
      grid=(N // _RB,),
      in_specs=[
          pl.BlockSpec((_RB, F), lambda i: (i, 0)),
          pl.BlockSpec((_RB, F), lambda i: (i, 0)),
          pl.BlockSpec((_RB, F), lambda i: (i, 0)),
          pl.BlockSpec((F, 3 * F), lambda i: (0, 0)),
          pl.BlockSpec((1, 3 * F), lambda i: (0, 0)),
      ],
      out_specs=pl.BlockSpec((_RB, 3 * F), lambda i: (i, 0)),
      out_shape=jax.ShapeDtypeStruct((N, 3 * F), jnp.float32),
  )(a, p0, p1, w, b)


def _add3(a, p0, p1):
  return pl.pallas_call(
      _add3_body,
      grid=(N // _RB,),
      in_specs=[pl.BlockSpec((_RB, F), lambda i: (i, 0))] * 3,
      out_specs=pl.BlockSpec((_RB, F), lambda i: (i, 0)),
      out_shape=jax.ShapeDtypeStruct((N, F), jnp.float32),
  )(a, p0, p1)


def kernel(x, edge_index, edge_weight, edge_index2, edge_weight2,
           ib1_ln_W, ib1_ln_b, ib1_c1_W, ib1_c1_b, ib1_c2_W, ib1_c2_b,
           ib2_ln_W, ib2_ln_b, ib2_c1_W, ib2_c1_b, ib2_c2_W, ib2_c2_b,
           ib3_ln_W, ib3_ln_b, ib3_c1_W, ib3_c1_b, ib3_c2_W, ib3_c2_b):
  s1 = edge_index[0].astype(jnp.int32)
  d1 = edge_index[1].astype(jnp.int32)
  s2 = edge_index2[0].astype(jnp.int32)
  d2 = edge_index2[1].astype(jnp.int32)
  w1 = edge_weight.astype(jnp.float32)
  w2 = edge_weight2.astype(jnp.float32)
  zeros = jnp.zeros((N, F), jnp.float32)

  def wcat(lw, lb, c1w, c1b, c2w, c2b):
    w = jnp.concatenate([lw, c1w, c2w], axis=1)
    b = jnp.concatenate([lb, c1b, c2b])[None, :]
    return w, b

  wc1, bc1 = wcat(ib1_ln_W, ib1_ln_b, ib1_c1_W, ib1_c1_b, ib1_c2_W, ib1_c2_b)
  wc2, bc2 = wcat(ib2_ln_W, ib2_ln_b, ib2_c1_W, ib2_c1_b, ib2_c2_W, ib2_c2_b)
  wc3, bc3 = wcat(ib3_ln_W, ib3_ln_b, ib3_c1_W, ib3_c1_b, ib3_c2_W, ib3_c2_b)

  t = _mm1(x, wc1, bc1)
  x0, z1, z2 = t[:, :F], t[:, F:2 * F], t[:, 2 * F:]
  p0, p1 = _spmm(z1, z2, s1, d1, w1, s2, d2, w2, zeros)

  t = _mm3(x0, p0, p1, wc2, bc2)
  x0, z1, z2 = t[:, :F], t[:, F:2 * F], t[:, 2 * F:]
  p0, p1 = _spmm(z1, z2, s1, d1, w1, s2, d2, w2, zeros)

  t = _mm3(x0, p0, p1, wc3, bc3)
  x0, z1, z2 = t[:, :F], t[:, F:2 * F], t[:, 2 * F:]
  p0, p1 = _spmm(z1, z2, s1, d1, w1, s2, d2, w2, zeros)

  return _add3(x0, p0, p1)


# speedup vs baseline: 3.4041x; 3.4041x over previous
import functools

import jax
import jax.numpy as jnp
from jax import lax
from jax.experimental import pallas as pl
from jax.experimental.pallas import tpu as pltpu
from jax.experimental.pallas import tpu_sc as plsc

N = 10000
F = 128
E1 = 320000
E2 = 640000
NC = 2
NS = 16
NW = NC * NS
CHUNK = 80


def _spmm_body(z1, z2, s1, d1, w1, s2, d2, w2, zeros,
               out0, out1, acc, idx_v, dst_v, w16_v, rows_v, sem):
  c = lax.axis_index("c")
  s = lax.axis_index("s")
  wid = c * NS + s

  @pl.when(s == 0)
  def _():
    pltpu.sync_copy(zeros, acc)

  plsc.subcore_barrier()

  def process(src_hbm, dst_hbm, ew_hbm, ztab, base, nchunks):
    def body(j, carry):
      off = pl.multiple_of(base + j * CHUNK, 8)
      pltpu.sync_copy(src_hbm.at[pl.ds(off, CHUNK)], idx_v)
      pltpu.sync_copy(dst_hbm.at[pl.ds(off, CHUNK)], dst_v)
      pltpu.sync_copy(ew_hbm.at[pl.ds(off, CHUNK)], w16_v)
      pltpu.async_copy(ztab.at[idx_v], rows_v, sem).wait()

      def scale(i, carry2):
        wb = w16_v[i, :]
        for jj in range(8):
          rows_v[i, pl.ds(jj * 16, 16)] = rows_v[i, pl.ds(jj * 16, 16)] * wb
        return carry2

      lax.fori_loop(0, CHUNK, scale, 0)
      pltpu.sync_copy(rows_v, acc.at[dst_v], add=True)
      return carry

    lax.fori_loop(0, nchunks, body, 0)

  process(s1, d1, w1, z1, wid * (E1 // NW), (E1 // NW) // CHUNK)
  process(s2, d2, w2, z2, wid * (E2 // NW), (E2 // NW) // CHUNK)

  plsc.subcore_barrier()

  @pl.when((s == 0) & (c == 0))
  def _():
    pltpu.sync_copy(acc, out0)

  @pl.when((s == 0) & (c == 1))
  def _():
    pltpu.sync_copy(acc, out1)


_spmm = functools.partial(
    pl.kernel,
    out_type=[jax.ShapeDtypeStruct((N, F), jnp.float32),
              jax.ShapeDtypeStruct((N, F), jnp.float32)],
    mesh=plsc.VectorSubcoreMesh(core_axis_name="c", subcore_axis_name="s"),
    scratch_types=[
        pltpu.VMEM_SHARED((N, F), jnp.float32),
        pltpu.VMEM((CHUNK,), jnp.int32),
        pltpu.VMEM((CHUNK,), jnp.int32),
        pltpu.VMEM((CHUNK, 16), jnp.float32),
        pltpu.VMEM((CHUNK, F), jnp.float32),
        pltpu.SemaphoreType.DMA,
    ],
)(_spmm_body)


_RB = 1000


def _mm1_body(x_ref, w_ref, b_ref, o_ref):
  o_ref[...] = lax.dot_general(
      x_ref[...], w_ref[...], (((1,), (0,)), ((), ())),
      preferred_element_type=jnp.float32,
      precision=lax.Precision.HIGHEST) + b_ref[...]


def _mm3_body(a_ref, p0_ref, p1_ref, w_ref, b_ref, o_ref):
  h = a_ref[...] + p0_ref[...] + p1_ref[...]
  o_ref[...] = lax.dot_general(
      h, w_ref[...], (((1,), (0,)), ((), ())),
      preferred_element_type=jnp.float32,
      precision=lax.Precision.HIGHEST) + b_ref[...]


def _add3_body(a_ref, p0_ref, p1_ref, o_ref):
  o_ref[...] = a_ref[...] + p0_ref[...] + p1_ref[...]


def _mm1(x, w, b):
  return pl.pallas_call(
      _mm1_body,
      grid=(N // _RB,),
      in_specs=[
          pl.BlockSpec((_RB, F), lambda i: (i, 0)),
          pl.BlockSpec((F, 3 * F), lambda i: (0, 0)),
          pl.BlockSpec((1, 3 * F), lambda i: (0, 0)),
      ],
      out_specs=pl.BlockSpec((_RB, 3 * F), lambda i: (i, 0)),
      out_shape=jax.ShapeDtypeStruct((N, 3 * F), jnp.float32),
  )(x, w, b)


def _mm3(a, p0, p1, w, b):
  return pl.pallas_call(
      _mm3_body,
      grid=(N // _RB,),
      in_specs=[
          pl.BlockSpec((_RB, F), lambda i: (i, 0)),
          pl.BlockSpec((_RB, F), lambda i: (i, 0)),
          pl.BlockSpec((_RB, F), lambda i: (i, 0)),
          pl.BlockSpec((F, 3 * F), lambda i: (0, 0)),
          pl.BlockSpec((1, 3 * F), lambda i: (0, 0)),
      ],
      out_specs=pl.BlockSpec((_RB, 3 * F), lambda i: (i, 0)),
      out_shape=jax.ShapeDtypeStruct((N, 3 * F), jnp.float32),
  )(a, p0, p1, w, b)


def _add3(a, p0, p1):
  return pl.pallas_call(
      _add3_body,
      grid=(N // _RB,),
      in_specs=[pl.BlockSpec((_RB, F), lambda i: (i, 0))] * 3,
      out_specs=pl.BlockSpec((_RB, F), lambda i: (i, 0)),
      out_shape=jax.ShapeDtypeStruct((N, F), jnp.float32),
  )(a, p0, p1)


def kernel(x, edge_index, edge_weight, edge_index2, edge_weight2,
           ib1_ln_W, ib1_ln_b, ib1_c1_W, ib1_c1_b, ib1_c2_W, ib1_c2_b,
           ib2_ln_W, ib2_ln_b, ib2_c1_W, ib2_c1_b, ib2_c2_W, ib2_c2_b,
           ib3_ln_W, ib3_ln_b, ib3_c1_W, ib3_c1_b, ib3_c2_W, ib3_c2_b):
  s1 = edge_index[0].astype(jnp.int32)
  d1 = edge_index[1].astype(jnp.int32)
  s2 = edge_index2[0].astype(jnp.int32)
  d2 = edge_index2[1].astype(jnp.int32)
  w1 = jnp.tile(edge_weight.astype(jnp.float32)[:, None], (1, 16))
  w2 = jnp.tile(edge_weight2.astype(jnp.float32)[:, None], (1, 16))
  zeros = jnp.zeros((N, F), jnp.float32)

  def wcat(lw, lb, c1w, c1b, c2w, c2b):
    w = jnp.concatenate([lw, c1w, c2w], axis=1)
    b = jnp.concatenate([lb, c1b, c2b])[None, :]
    return w, b

  wc1, bc1 = wcat(ib1_ln_W, ib1_ln_b, ib1_c1_W, ib1_c1_b, ib1_c2_W, ib1_c2_b)
  wc2, bc2 = wcat(ib2_ln_W, ib2_ln_b, ib2_c1_W, ib2_c1_b, ib2_c2_W, ib2_c2_b)
  wc3, bc3 = wcat(ib3_ln_W, ib3_ln_b, ib3_c1_W, ib3_c1_b, ib3_c2_W, ib3_c2_b)

  t = _mm1(x, wc1, bc1)
  x0, z1, z2 = t[:, :F], t[:, F:2 * F], t[:, 2 * F:]
  p0, p1 = _spmm(z1, z2, s1, d1, w1, s2, d2, w2, zeros)

  t = _mm3(x0, p0, p1, wc2, bc2)
  x0, z1, z2 = t[:, :F], t[:, F:2 * F], t[:, 2 * F:]
  p0, p1 = _spmm(z1, z2, s1, d1, w1, s2, d2, w2, zeros)

  t = _mm3(x0, p0, p1, wc3, bc3)
  x0, z1, z2 = t[:, :F], t[:, F:2 * F], t[:, 2 * F:]
  p0, p1 = _spmm(z1, z2, s1, d1, w1, s2, d2, w2, zeros)

  return _add3(x0, p0, p1)
